# baseline (speedup 1.0000x reference)
import functools

import jax
import jax.numpy as jnp
from jax import lax
from jax.experimental import pallas as pl
from jax.experimental.pallas import tpu as pltpu
from jax.experimental.pallas import tpu_sc as plsc

R = 2048
N = 8192
K = N - 1 - N // 2
NW = 32
ROWS_PER_W = R // NW
L = 16

_INT_MIN_PY = -2147483648


def _make_sc_kernel():
    mesh = plsc.VectorSubcoreMesh(core_axis_name="c", subcore_axis_name="s")

    @functools.partial(
        pl.kernel,
        mesh=mesh,
        compiler_params=pltpu.CompilerParams(needs_layout_passes=False),
        out_type=jax.ShapeDtypeStruct((R,), jnp.float32),
        scratch_types=[
            pltpu.VMEM((N,), jnp.float32),
            pltpu.VMEM((3 * N,), jnp.int32),
            pltpu.VMEM((ROWS_PER_W,), jnp.float32),
        ],
    )
    def sc_kernel(x_hbm, out_hbm, row_v, keys_v, res_v):
        wid = lax.axis_index("c") * 16 + lax.axis_index("s")
        lane = lax.iota(jnp.int32, L)
        zero16 = jnp.zeros((L,), jnp.int32)

        def row_body(r, carry):
            grow = wid * ROWS_PER_W + r
            pltpu.sync_copy(x_hbm.at[grow], row_v)

            def fbody(j, vc):
                accv, vA, vB = vc
                xv = row_v[pl.ds(j * L, L)]
                iv = plsc.bitcast(xv, jnp.int32)
                ks = iv ^ ((iv >> 31) & jnp.int32(0x7FFFFFFF))
                ml = ks < jnp.int32(0)
                mh = ~ml
                il = ml.astype(jnp.int32)
                ih = mh.astype(jnp.int32)
                posA = vA + plsc.cumsum(il) - 1
                posB = vB + plsc.cumsum(ih) - 1
                plsc.store_scatter(keys_v, [posA], ks, mask=ml)
                plsc.store_scatter(keys_v, [posB], ks, mask=mh)
                return (
                    accv + il,
                    vA + plsc.all_reduce_population_count(ml),
                    vB + plsc.all_reduce_population_count(mh),
                )

            accv, _, _ = lax.fori_loop(
                0, N // L, fbody,
                (zero16, zero16, jnp.full((L,), N, jnp.int32)),
            )
            c = jnp.sum(accv)
            low0 = K < c
            p = lax.select(low0, jnp.int32(0), jnp.int32(_INT_MIN_PY))
            k = lax.select(low0, jnp.int32(K), jnp.int32(K) - c)
            n = lax.select(low0, c, jnp.int32(N) - c)
            sb = lax.select(low0, jnp.int32(0), jnp.int32(N))
            ab = jnp.int32(2 * N)
            bb = lax.select(low0, jnp.int32(N), jnp.int32(0))

            def bit_body(t, bc):
                p, k, n, sb, ab, bb = bc
                cand = p | lax.shift_left(jnp.int32(1), jnp.int32(30) - t)
                ccmp = cand ^ jnp.int32(_INT_MIN_PY)

                def vbody(j, vc):
                    accv, vA, vB = vc
                    ks = keys_v[pl.ds(sb + j * L, L)]
                    valid = (j * L + lane) < n
                    ml = (ks < ccmp) & valid
                    mh = valid & ~ml
                    il = ml.astype(jnp.int32)
                    ih = mh.astype(jnp.int32)
                    posA = vA + plsc.cumsum(il) - 1
                    posB = vB + plsc.cumsum(ih) - 1
                    plsc.store_scatter(keys_v, [posA], ks, mask=ml)
                    plsc.store_scatter(keys_v, [posB], ks, mask=mh)
                    return (
                        accv + il,
                        vA + plsc.all_reduce_population_count(ml),
                        vB + plsc.all_reduce_population_count(mh),
                    )

                nv = (n + L - 1) // L
                accv, _, _ = lax.fori_loop(
                    0, nv, vbody,
                    (zero16,
                     jnp.full((L,), ab, jnp.int32),
                     jnp.full((L,), bb, jnp.int32)),
                )
                c = jnp.sum(accv)
                low = k < c
                p2 = lax.select(low, p, cand)
                k2 = lax.select(low, k, k - c)
                n2 = lax.select(low, c, n - c)
                sb2 = lax.select(low, ab, bb)
                bb2 = lax.select(low, bb, ab)
                return p2, k2, n2, sb2, sb, bb2

            p, _, _, _, _, _ = lax.fori_loop(
                0, 31, bit_body, (p, k, n, sb, ab, bb)
            )

            pos = p < jnp.int32(0)
            fbits = lax.select(pos, p ^ jnp.int32(_INT_MIN_PY), ~p)
            val = lax.bitcast_convert_type(fbits, jnp.float32)

            plsc.store_scatter(
                res_v,
                [jnp.full((L,), r, jnp.int32)],
                jnp.full((L,), val, jnp.float32),
                mask=lane == jnp.int32(0),
            )
            return carry

        lax.fori_loop(0, ROWS_PER_W, row_body, jnp.int32(0))
        pltpu.sync_copy(res_v, out_hbm.at[pl.ds(wid * ROWS_PER_W, ROWS_PER_W)])

    return sc_kernel


_sc_kernel = _make_sc_kernel()


def kernel(x):
    B0, B1, n = x.shape
    flat = _sc_kernel(x.reshape(B0 * B1, n))
    return flat.reshape(B0, B1)


# speedup vs baseline: 1.4044x; 1.4044x over previous
import functools

import jax
import jax.numpy as jnp
from jax import lax
from jax.experimental import pallas as pl
from jax.experimental.pallas import tpu as pltpu
from jax.experimental.pallas import tpu_sc as plsc

R = 2048
N = 8192
K = N - 1 - N // 2
NW = 32
RPW = R // NW
L = 16
NV = N // L
U = 4
NB = 64
SH = 26
HIST_W = U * L * NB

_INT_MIN_PY = -2147483648


def _make_sc_kernel():
    mesh = plsc.VectorSubcoreMesh(core_axis_name="c", subcore_axis_name="s")

    @functools.partial(
        pl.kernel,
        mesh=mesh,
        compiler_params=pltpu.CompilerParams(needs_layout_passes=False),
        out_type=jax.ShapeDtypeStruct((R,), jnp.float32),
        scratch_types=[
            pltpu.VMEM((2 * N,), jnp.float32),
            pltpu.VMEM((3 * N,), jnp.int32),
            pltpu.VMEM((HIST_W,), jnp.int32),
            pltpu.VMEM((RPW,), jnp.float32),
            pltpu.SemaphoreType.DMA,
        ],
    )
    def sc_kernel(x_hbm, out_hbm, rows_v, keys_v, hist_v, res_v, sem):
        wid = lax.axis_index("c") * 16 + lax.axis_index("s")
        base0 = wid * RPW
        lane = lax.iota(jnp.int32, L)
        zero16 = jnp.zeros((L,), jnp.int32)
        ones16 = jnp.ones((L,), jnp.int32)
        int_min = jnp.int32(_INT_MIN_PY)
        lane_nb = lane * NB

        def clr(i, c):
            hist_v[pl.ds(i * L, L)] = zero16
            return c

        lax.fori_loop(0, HIST_W // L, clr, 0)

        pltpu.async_copy(x_hbm.at[base0], rows_v.at[pl.ds(0, N)], sem)

        def load_key(off):
            xv = rows_v[pl.ds(off, L)]
            iv = plsc.bitcast(xv, jnp.int32)
            return iv ^ ((iv >> 31) & jnp.int32(0x7FFFFFFF))

        def row_body(r, carry):
            nxt = r + 1

            @pl.when(nxt < RPW)
            def _():
                pltpu.async_copy(
                    x_hbm.at[base0 + nxt],
                    rows_v.at[pl.ds((nxt % 2) * N, N)],
                    sem,
                )

            pltpu.make_async_copy(
                x_hbm.at[base0], rows_v.at[pl.ds(0, N)], sem
            ).wait()
            rb = (r % 2) * N

            def h_body(j, c):
                for u in range(U):
                    ks = load_key(rb + (j * U + u) * L)
                    du = ((ks ^ int_min) >> SH) & jnp.int32(NB - 1)
                    idx = lane_nb + (u * L * NB) + du
                    plsc.addupdate_scatter(hist_v, [idx], ones16)
                return c

            lax.fori_loop(0, NV // U, h_body, 0)

            tot = []
            for i in range(NB // L):
                acc = zero16
                for s_ in range(U * L):
                    sl = pl.ds(s_ * NB + i * L, L)
                    acc = acc + hist_v[sl]
                    hist_v[sl] = zero16
                tot.append(acc)

            kk = jnp.int32(K)
            t0 = jnp.sum(tot[0])
            t1 = jnp.sum(tot[1])
            t2 = jnp.sum(tot[2])
            c1 = t0
            c2 = c1 + t1
            c3 = c2 + t2
            i_star = (
                (kk >= c1).astype(jnp.int32)
                + (kk >= c2).astype(jnp.int32)
                + (kk >= c3).astype(jnp.int32)
            )
            tb = jnp.where(
                kk >= c3, c3, jnp.where(kk >= c2, c2,
                                        jnp.where(kk >= c1, c1, jnp.int32(0)))
            )
            pv = jnp.full((L,), i_star, jnp.int32)
            tot_sel = jnp.where(
                pv == 0, tot[0],
                jnp.where(pv == 1, tot[1], jnp.where(pv == 2, tot[2], tot[3])),
            )
            cum = plsc.cumsum(tot_sel) + tb
            mle = cum <= kk
            d_vec = plsc.all_reduce_population_count(mle) + i_star * L
            cum_before = jnp.max(jnp.where(mle, cum, tb))
            cum_d = jnp.min(jnp.where(mle, jnp.int32(1 << 30), cum))
            n = cum_d - cum_before
            k = kk - cum_before
            d_scalar = jnp.max(d_vec)
            p = lax.shift_left(d_scalar, jnp.int32(SH))

            def c_body(j, vw):
                for u in range(U):
                    ks = load_key(rb + (j * U + u) * L)
                    du = ((ks ^ int_min) >> SH) & jnp.int32(NB - 1)
                    m = du == d_vec
                    pos = vw + plsc.cumsum(m.astype(jnp.int32)) - 1
                    plsc.store_scatter(keys_v, [pos], ks, mask=m)
                    vw = vw + plsc.all_reduce_population_count(m)
                return vw

            lax.fori_loop(0, NV // U, c_body, zero16)

            def bit_body(t, bc):
                p, k, n, sb, ab, bb = bc
                cand = p | lax.shift_left(jnp.int32(1), jnp.int32(SH - 1) - t)
                ccmp = cand ^ int_min

                def vbody(j, vc):
                    accv, va, vb = vc
                    ks = keys_v[pl.ds(sb + j * L, L)]
                    valid = (j * L + lane) < n
                    ml = (ks < ccmp) & valid
                    mh = valid & ~ml
                    il = ml.astype(jnp.int32)
                    posa = va + plsc.cumsum(il) - 1
                    posb = vb + plsc.cumsum(mh.astype(jnp.int32)) - 1
                    plsc.store_scatter(keys_v, [posa], ks, mask=ml)
                    plsc.store_scatter(keys_v, [posb], ks, mask=mh)
                    return (
                        accv + il,
                        va + plsc.all_reduce_population_count(ml),
                        vb + plsc.all_reduce_population_count(mh),
                    )

                nv = (n + L - 1) // L
                accv, _, _ = lax.fori_loop(
                    0, nv, vbody,
                    (zero16,
                     jnp.full((L,), ab, jnp.int32),
                     jnp.full((L,), bb, jnp.int32)),
                )
                c = jnp.sum(accv)
                low = k < c
                p2 = lax.select(low, p, cand)
                k2 = lax.select(low, k, k - c)
                n2 = lax.select(low, c, n - c)
                sb2 = lax.select(low, ab, bb)
                bb2 = lax.select(low, bb, ab)
                return p2, k2, n2, sb2, sb, bb2

            p, _, _, _, _, _ = lax.fori_loop(
                0, SH, bit_body,
                (p, k, n, jnp.int32(0), jnp.int32(N), jnp.int32(2 * N)),
            )

            pos = p < jnp.int32(0)
            fbits = lax.select(pos, p ^ int_min, ~p)
            val = lax.bitcast_convert_type(fbits, jnp.float32)

            plsc.store_scatter(
                res_v,
                [jnp.full((L,), r, jnp.int32)],
                jnp.full((L,), val, jnp.float32),
                mask=lane == jnp.int32(0),
            )
            return carry

        lax.fori_loop(0, RPW, row_body, jnp.int32(0))
        pltpu.sync_copy(res_v, out_hbm.at[pl.ds(wid * RPW, RPW)])

    return sc_kernel


_sc_kernel = _make_sc_kernel()


def kernel(x):
    B0, B1, n = x.shape
    flat = _sc_kernel(x.reshape(B0 * B1, n))
    return flat.reshape(B0, B1)


# speedup vs baseline: 4.7324x; 3.3697x over previous
import functools

import jax
import jax.numpy as jnp
from jax import lax
from jax.experimental import pallas as pl
from jax.experimental.pallas import tpu as pltpu
from jax.experimental.pallas import tpu_sc as plsc

R = 2048
N = 8192
K = N - 1 - N // 2
NW = 32
RPW = R // NW
L = 16
NV = N // L
U = 4
NB = 64
SH = 26
HIST_W = U * L * NB

_INT_MIN_PY = -2147483648


def _make_sc_kernel():
    mesh = plsc.VectorSubcoreMesh(core_axis_name="c", subcore_axis_name="s")

    @functools.partial(
        pl.kernel,
        mesh=mesh,
        compiler_params=pltpu.CompilerParams(needs_layout_passes=False),
        out_type=jax.ShapeDtypeStruct((R,), jnp.float32),
        scratch_types=[
            pltpu.VMEM((2 * N,), jnp.float32),
            pltpu.VMEM((3 * N,), jnp.int32),
            pltpu.VMEM((HIST_W,), jnp.int32),
            pltpu.VMEM((RPW,), jnp.float32),
            pltpu.SemaphoreType.DMA,
        ],
    )
    def sc_kernel(x_hbm, out_hbm, rows_v, keys_v, hist_v, res_v, sem):
        wid = lax.axis_index("c") * 16 + lax.axis_index("s")
        base0 = wid * RPW
        lane = lax.iota(jnp.int32, L)
        zero16 = jnp.zeros((L,), jnp.int32)
        ones16 = jnp.ones((L,), jnp.int32)
        int_min = jnp.int32(_INT_MIN_PY)
        lane_nb = lane * NB

        @plsc.parallel_loop(0, HIST_W // L, unroll=4)
        def _(i):
            hist_v[pl.ds(i * L, L)] = zero16

        pltpu.async_copy(x_hbm.at[base0], rows_v.at[pl.ds(0, N)], sem)

        def load_key(off):
            xv = rows_v[pl.ds(off, L)]
            iv = plsc.bitcast(xv, jnp.int32)
            return iv ^ ((iv >> 31) & jnp.int32(0x7FFFFFFF))

        def row_body(r, carry):
            nxt = r + 1

            @pl.when(nxt < RPW)
            def _():
                pltpu.async_copy(
                    x_hbm.at[base0 + nxt],
                    rows_v.at[pl.ds((nxt % 2) * N, N)],
                    sem,
                )

            pltpu.make_async_copy(
                x_hbm.at[base0], rows_v.at[pl.ds(0, N)], sem
            ).wait()
            rb = (r % 2) * N

            @plsc.parallel_loop(0, NV, unroll=U)
            def _(i):
                ks = load_key(rb + i * L)
                du = ((ks ^ int_min) >> SH) & jnp.int32(NB - 1)
                idx = lane_nb + (i % U) * (L * NB) + du
                plsc.addupdate_scatter(hist_v, [idx], ones16)

            def tot_body(s_, tc):
                outs = []
                for i in range(NB // L):
                    sl = pl.ds(s_ * NB + i * L, L)
                    outs.append(tc[i] + hist_v[sl])
                    hist_v[sl] = zero16
                return tuple(outs)

            tot = plsc.parallel_loop(
                0, U * L, unroll=2,
                carry=(zero16, zero16, zero16, zero16),
            )(tot_body)

            kk = jnp.int32(K)
            t0 = jnp.sum(tot[0])
            t1 = jnp.sum(tot[1])
            t2 = jnp.sum(tot[2])
            c1 = t0
            c2 = c1 + t1
            c3 = c2 + t2
            i_star = (
                (kk >= c1).astype(jnp.int32)
                + (kk >= c2).astype(jnp.int32)
                + (kk >= c3).astype(jnp.int32)
            )
            tb = jnp.where(
                kk >= c3, c3, jnp.where(kk >= c2, c2,
                                        jnp.where(kk >= c1, c1, jnp.int32(0)))
            )
            pv = jnp.full((L,), i_star, jnp.int32)
            tot_sel = jnp.where(
                pv == 0, tot[0],
                jnp.where(pv == 1, tot[1], jnp.where(pv == 2, tot[2], tot[3])),
            )
            cum = plsc.cumsum(tot_sel) + tb
            mle = cum <= kk
            d_vec = plsc.all_reduce_population_count(mle) + i_star * L
            cum_before = jnp.max(jnp.where(mle, cum, tb))
            cum_d = jnp.min(jnp.where(mle, jnp.int32(1 << 30), cum))
            n = cum_d - cum_before
            k = kk - cum_before
            d_scalar = jnp.max(d_vec)
            p = lax.shift_left(d_scalar, jnp.int32(SH))

            def compact_body(i, vw):
                ks = load_key(rb + i * L)
                du = ((ks ^ int_min) >> SH) & jnp.int32(NB - 1)
                m = du == d_vec
                pos = vw + plsc.cumsum(m.astype(jnp.int32)) - 1
                plsc.store_scatter(keys_v, [pos], ks, mask=m)
                return vw + plsc.all_reduce_population_count(m)

            plsc.parallel_loop(0, NV, unroll=U, carry=zero16)(compact_body)

            def level_cond(bc):
                _, _, n, _, _, _, b = bc
                return (n > L) & (b >= 0)

            def level(bc):
                p, k, n, sb, ab, bb, b = bc
                cand = p | lax.shift_left(jnp.int32(1), b)
                ccmp = cand ^ int_min
                nv = (n + L - 1) // L

                def level_pass(j, vc):
                    accv, va, vb = vc
                    ks = keys_v[pl.ds(sb + j * L, L)]
                    valid = (j * L + lane) < n
                    ml = (ks < ccmp) & valid
                    mh = valid & ~ml
                    il = ml.astype(jnp.int32)
                    posa = va + plsc.cumsum(il) - 1
                    posb = vb + plsc.cumsum(mh.astype(jnp.int32)) - 1
                    plsc.store_scatter(keys_v, [posa], ks, mask=ml)
                    plsc.store_scatter(keys_v, [posb], ks, mask=mh)
                    return (
                        accv + il,
                        va + plsc.all_reduce_population_count(ml),
                        vb + plsc.all_reduce_population_count(mh),
                    )

                acc = plsc.parallel_loop(
                    0, nv, unroll=2,
                    carry=(zero16,
                           jnp.full((L,), ab, jnp.int32),
                           jnp.full((L,), bb, jnp.int32)),
                )(level_pass)
                c = jnp.sum(acc[0])
                low = k < c
                p2 = lax.select(low, p, cand)
                k2 = lax.select(low, k, k - c)
                n2 = lax.select(low, c, n - c)
                sb2 = lax.select(low, ab, bb)
                bb2 = lax.select(low, bb, ab)
                return p2, k2, n2, sb2, sb, bb2, b - 1

            p, k, n, sb, _, _, b = lax.while_loop(
                level_cond, level,
                (p, k, n, jnp.int32(0), jnp.int32(N), jnp.int32(2 * N),
                 jnp.int32(SH - 1)),
            )

            def tail():
                ks = keys_v[pl.ds(sb, L)]
                ks = jnp.where(lane < n, ks, jnp.int32(0x7FFFFFFF))
                srt = lax.sort(ks)
                kv = jnp.take_along_axis(
                    srt, jnp.full((L,), k, jnp.int32), axis=0,
                    mode="promise_in_bounds",
                )
                return plsc.bitcast(
                    kv ^ ((kv >> 31) & jnp.int32(0x7FFFFFFF)), jnp.float32
                )

            def from_prefix():
                pos = p < jnp.int32(0)
                fbits = lax.select(pos, p ^ int_min, ~p)
                return jnp.full(
                    (L,), lax.bitcast_convert_type(fbits, jnp.float32)
                )

            val = lax.cond(n <= L, tail, from_prefix)

            plsc.store_scatter(
                res_v,
                [jnp.full((L,), r, jnp.int32)],
                val,
                mask=lane == jnp.int32(0),
            )
            return carry

        lax.fori_loop(0, RPW, row_body, jnp.int32(0))
        pltpu.sync_copy(res_v, out_hbm.at[pl.ds(wid * RPW, RPW)])

    return sc_kernel


_sc_kernel = _make_sc_kernel()


def kernel(x):
    B0, B1, n = x.shape
    flat = _sc_kernel(x.reshape(B0 * B1, n))
    return flat.reshape(B0, B1)


# speedup vs baseline: 6.4609x; 1.3652x over previous
import functools

import jax
import jax.numpy as jnp
from jax import lax
from jax.experimental import pallas as pl
from jax.experimental.pallas import tpu as pltpu
from jax.experimental.pallas import tpu_sc as plsc

R = 2048
N = 8192
K = N - 1 - N // 2
NW = 32
RPW = R // NW
L = 16
NV = N // L
U = 4
NB = 256
SH = 24
HIST_W = U * L * NB

_INT_MIN_PY = -2147483648


def _make_sc_kernel():
    mesh = plsc.VectorSubcoreMesh(core_axis_name="c", subcore_axis_name="s")

    @functools.partial(
        pl.kernel,
        mesh=mesh,
        compiler_params=pltpu.CompilerParams(needs_layout_passes=False),
        out_type=jax.ShapeDtypeStruct((R,), jnp.float32),
        scratch_types=[
            pltpu.VMEM((2 * N,), jnp.float32),
            pltpu.VMEM((3 * N,), jnp.int32),
            pltpu.VMEM((HIST_W,), jnp.int32),
            pltpu.VMEM((RPW,), jnp.float32),
            pltpu.SemaphoreType.DMA,
        ],
    )
    def sc_kernel(x_hbm, out_hbm, rows_v, keys_v, hist_v, res_v, sem):
        wid = lax.axis_index("c") * 16 + lax.axis_index("s")
        base0 = wid * RPW
        lane = lax.iota(jnp.int32, L)
        zero16 = jnp.zeros((L,), jnp.int32)
        ones16 = jnp.ones((L,), jnp.int32)
        int_min = jnp.int32(_INT_MIN_PY)
        lane_nb = lane * NB

        @plsc.parallel_loop(0, HIST_W // L, unroll=4)
        def _(i):
            hist_v[pl.ds(i * L, L)] = zero16

        pltpu.async_copy(x_hbm.at[base0], rows_v.at[pl.ds(0, N)], sem)

        def load_key(off):
            xv = rows_v[pl.ds(off, L)]
            iv = plsc.bitcast(xv, jnp.int32)
            return iv ^ ((iv >> 31) & jnp.int32(0x7FFFFFFF))

        def row_body(r, carry):
            nxt = r + 1

            @pl.when(nxt < RPW)
            def _():
                pltpu.async_copy(
                    x_hbm.at[base0 + nxt],
                    rows_v.at[pl.ds((nxt % 2) * N, N)],
                    sem,
                )

            pltpu.make_async_copy(
                x_hbm.at[base0], rows_v.at[pl.ds(0, N)], sem
            ).wait()
            rb = (r % 2) * N

            @plsc.parallel_loop(0, NV, unroll=U)
            def _(i):
                ks = load_key(rb + i * L)
                du = ((ks ^ int_min) >> SH) & jnp.int32(NB - 1)
                idx = lane_nb + (i % U) * (L * NB) + du
                plsc.addupdate_scatter(hist_v, [idx], ones16)

            NG = NB // L

            def tot_body(s_, tc):
                outs = []
                for i in range(NG):
                    sl = pl.ds(s_ * NB + i * L, L)
                    outs.append(tc[i] + hist_v[sl])
                    hist_v[sl] = zero16
                return tuple(outs)

            tot = plsc.parallel_loop(
                0, U * L, unroll=1, carry=(zero16,) * NG
            )(tot_body)

            kk = jnp.int32(K)
            t = [jnp.sum(tot[i]) for i in range(NG)]
            cums = []
            run = t[0]
            for i in range(1, NG):
                cums.append(run)
                run = run + t[i]
            i_star = jnp.int32(0)
            for c in cums:
                i_star = i_star + (kk >= c).astype(jnp.int32)
            tb = jnp.int32(0)
            for c in cums:
                tb = jnp.where(kk >= c, c, tb)
            pv = jnp.full((L,), i_star, jnp.int32)
            tot_sel = tot[NG - 1]
            for i in range(NG - 2, -1, -1):
                tot_sel = jnp.where(pv == i, tot[i], tot_sel)
            cum = plsc.cumsum(tot_sel) + tb
            mle = cum <= kk
            d_vec = plsc.all_reduce_population_count(mle) + i_star * L
            cum_before = jnp.max(jnp.where(mle, cum, tb))
            cum_d = jnp.min(jnp.where(mle, jnp.int32(1 << 30), cum))
            n = cum_d - cum_before
            k = kk - cum_before
            d_scalar = jnp.max(d_vec)
            p = lax.shift_left(d_scalar, jnp.int32(SH))

            def compact_body(i, vw):
                ks = load_key(rb + i * L)
                du = ((ks ^ int_min) >> SH) & jnp.int32(NB - 1)
                m = du == d_vec
                pos = vw + plsc.cumsum(m.astype(jnp.int32)) - 1
                plsc.store_scatter(keys_v, [pos], ks, mask=m)
                return vw + plsc.all_reduce_population_count(m)

            plsc.parallel_loop(0, NV, unroll=U, carry=zero16)(compact_body)

            sb0 = jnp.int32(0)
            ab0 = jnp.int32(N)
            bb0 = jnp.int32(2 * N)
            b0 = jnp.int32(SH - 1)

            def level_cond(bc):
                _, _, n, _, _, _, b = bc
                return (n > L) & (b >= 0)

            def level(bc):
                p, k, n, sb, ab, bb, b = bc
                cand = p | lax.shift_left(jnp.int32(1), b)
                ccmp = cand ^ int_min
                nv = (n + L - 1) // L

                def level_pass(j, vc):
                    accv, va, vb = vc
                    ks = keys_v[pl.ds(sb + j * L, L)]
                    valid = (j * L + lane) < n
                    ml = (ks < ccmp) & valid
                    mh = valid & ~ml
                    il = ml.astype(jnp.int32)
                    posa = va + plsc.cumsum(il) - 1
                    posb = vb + plsc.cumsum(mh.astype(jnp.int32)) - 1
                    plsc.store_scatter(keys_v, [posa], ks, mask=ml)
                    plsc.store_scatter(keys_v, [posb], ks, mask=mh)
                    return (
                        accv + il,
                        va + plsc.all_reduce_population_count(ml),
                        vb + plsc.all_reduce_population_count(mh),
                    )

                acc = plsc.parallel_loop(
                    0, nv, unroll=2,
                    carry=(zero16,
                           jnp.full((L,), ab, jnp.int32),
                           jnp.full((L,), bb, jnp.int32)),
                )(level_pass)
                c = jnp.sum(acc[0])
                low = k < c
                p2 = lax.select(low, p, cand)
                k2 = lax.select(low, k, k - c)
                n2 = lax.select(low, c, n - c)
                sb2 = lax.select(low, ab, bb)
                bb2 = lax.select(low, bb, ab)
                return p2, k2, n2, sb2, sb, bb2, b - 1

            p, k, n, sb, _, _, b = lax.while_loop(
                level_cond, level, (p, k, n, sb0, ab0, bb0, b0)
            )

            def tail():
                ks = keys_v[pl.ds(sb, L)]
                ks = jnp.where(lane < n, ks, jnp.int32(0x7FFFFFFF))
                srt = lax.sort(ks)
                kv = jnp.take_along_axis(
                    srt, jnp.full((L,), k, jnp.int32), axis=0,
                    mode="promise_in_bounds",
                )
                return plsc.bitcast(
                    kv ^ ((kv >> 31) & jnp.int32(0x7FFFFFFF)), jnp.float32
                )

            def from_prefix():
                pos = p < jnp.int32(0)
                fbits = lax.select(pos, p ^ int_min, ~p)
                return jnp.full(
                    (L,), lax.bitcast_convert_type(fbits, jnp.float32)
                )

            val = lax.cond(n <= L, tail, from_prefix)

            plsc.store_scatter(
                res_v,
                [jnp.full((L,), r, jnp.int32)],
                val,
                mask=lane == jnp.int32(0),
            )
            return carry

        lax.fori_loop(0, RPW, row_body, jnp.int32(0))
        pltpu.sync_copy(res_v, out_hbm.at[pl.ds(wid * RPW, RPW)])

    return sc_kernel


_sc_kernel = _make_sc_kernel()


def kernel(x):
    B0, B1, n = x.shape
    flat = _sc_kernel(x.reshape(B0 * B1, n))
    return flat.reshape(B0, B1)


# speedup vs baseline: 6.7713x; 1.0480x over previous
import functools

import jax
import jax.numpy as jnp
from jax import lax
from jax.experimental import pallas as pl
from jax.experimental.pallas import tpu as pltpu
from jax.experimental.pallas import tpu_sc as plsc

R = 2048
N = 8192
K = N - 1 - N // 2
NW = 32
RPW = R // NW
L = 16
NV = N // L
U = 4
NB = 256
SH = 24
HIST_W = U * L * NB

_INT_MIN_PY = -2147483648


def _make_sc_kernel():
    mesh = plsc.VectorSubcoreMesh(core_axis_name="c", subcore_axis_name="s")

    @functools.partial(
        pl.kernel,
        mesh=mesh,
        compiler_params=pltpu.CompilerParams(needs_layout_passes=False),
        out_type=jax.ShapeDtypeStruct((R,), jnp.float32),
        scratch_types=[
            pltpu.VMEM((2 * N,), jnp.float32),
            pltpu.VMEM((3 * N,), jnp.int32),
            pltpu.VMEM((HIST_W,), jnp.int32),
            pltpu.VMEM((RPW,), jnp.float32),
            pltpu.SemaphoreType.DMA,
        ],
    )
    def sc_kernel(x_hbm, out_hbm, rows_v, keys_v, hist_v, res_v, sem):
        wid = lax.axis_index("c") * 16 + lax.axis_index("s")
        base0 = wid * RPW
        lane = lax.iota(jnp.int32, L)
        zero16 = jnp.zeros((L,), jnp.int32)
        ones16 = jnp.ones((L,), jnp.int32)
        int_min = jnp.int32(_INT_MIN_PY)
        lane_nb = lane * NB

        @plsc.parallel_loop(0, HIST_W // L, unroll=4)
        def _(i):
            hist_v[pl.ds(i * L, L)] = zero16

        pltpu.async_copy(x_hbm.at[base0], rows_v.at[pl.ds(0, N)], sem)

        def load_key(off):
            xv = rows_v[pl.ds(off, L)]
            iv = plsc.bitcast(xv, jnp.int32)
            return iv ^ ((iv >> 31) & jnp.int32(0x7FFFFFFF))

        def row_body(r, carry):
            nxt = r + 1

            @pl.when(nxt < RPW)
            def _():
                pltpu.async_copy(
                    x_hbm.at[base0 + nxt],
                    rows_v.at[pl.ds((nxt % 2) * N, N)],
                    sem,
                )

            pltpu.make_async_copy(
                x_hbm.at[base0], rows_v.at[pl.ds(0, N)], sem
            ).wait()
            rb = (r % 2) * N

            @plsc.parallel_loop(0, NV, unroll=U)
            def _(i):
                ks = load_key(rb + i * L)
                du = ((ks ^ int_min) >> SH) & jnp.int32(NB - 1)
                idx = lane_nb + (i % U) * (L * NB) + du
                plsc.addupdate_scatter(hist_v, [idx], ones16)

            NG = NB // L

            def tot_body(s_, tc):
                outs = []
                for i in range(NG):
                    sl = pl.ds(s_ * NB + i * L, L)
                    outs.append(tc[i] + hist_v[sl])
                    hist_v[sl] = zero16
                return tuple(outs)

            tot = plsc.parallel_loop(
                0, U * L, unroll=1, carry=(zero16,) * NG
            )(tot_body)

            kk = jnp.int32(K)
            t = [jnp.sum(tot[i]) for i in range(NG)]
            cums = []
            run = t[0]
            for i in range(1, NG):
                cums.append(run)
                run = run + t[i]
            i_star = jnp.int32(0)
            for c in cums:
                i_star = i_star + (kk >= c).astype(jnp.int32)
            tb = jnp.int32(0)
            for c in cums:
                tb = jnp.where(kk >= c, c, tb)
            pv = jnp.full((L,), i_star, jnp.int32)
            tot_sel = tot[NG - 1]
            for i in range(NG - 2, -1, -1):
                tot_sel = jnp.where(pv == i, tot[i], tot_sel)
            cum = plsc.cumsum(tot_sel) + tb
            mle = cum <= kk
            d_vec = plsc.all_reduce_population_count(mle) + i_star * L
            cum_before = jnp.max(jnp.where(mle, cum, tb))
            cum_d = jnp.min(jnp.where(mle, jnp.int32(1 << 30), cum))
            n = cum_d - cum_before
            k = kk - cum_before
            d_scalar = jnp.max(d_vec)
            p = lax.shift_left(d_scalar, jnp.int32(SH))

            dge = d_scalar >= jnp.int32(NB // 2)
            a_s = jnp.where(
                dge,
                lax.shift_left(d_scalar - jnp.int32(NB // 2), jnp.int32(SH)),
                -lax.shift_left(d_scalar + jnp.int32(1), jnp.int32(SH)),
            )
            b_s = jnp.where(
                dge,
                lax.shift_left(d_scalar - jnp.int32(NB // 2 - 1), jnp.int32(SH)),
                -lax.shift_left(d_scalar, jnp.int32(SH)),
            )
            a_v = jnp.full((L,), a_s, jnp.int32)
            b_v = jnp.full((L,), b_s, jnp.int32)

            def compact_body(i, vw):
                xv = rows_v[pl.ds(rb + i * L, L)]
                iv = plsc.bitcast(xv, jnp.int32)
                m = (iv >= a_v) & (iv < b_v)
                pos = vw + plsc.cumsum(m.astype(jnp.int32)) - 1
                plsc.store_scatter(keys_v, [pos], iv, mask=m)
                return vw + plsc.all_reduce_population_count(m)

            plsc.parallel_loop(0, NV, unroll=U, carry=zero16)(compact_body)

            sb0 = jnp.int32(0)
            ab0 = jnp.int32(N)
            bb0 = jnp.int32(2 * N)
            b0 = jnp.int32(SH - 1)

            def level_cond(bc):
                _, _, n, _, _, _, b = bc
                return (n > L) & (b >= 0)

            def level(bc):
                p, k, n, sb, ab, bb, b = bc
                cand = p | lax.shift_left(jnp.int32(1), b)
                ccmp = cand ^ int_min
                nv = (n + L - 1) // L

                def level_pass(j, vc):
                    accv, va, vb = vc
                    iv = keys_v[pl.ds(sb + j * L, L)]
                    ks = iv ^ ((iv >> 31) & jnp.int32(0x7FFFFFFF))
                    valid = (j * L + lane) < n
                    ml = (ks < ccmp) & valid
                    mh = valid & ~ml
                    il = ml.astype(jnp.int32)
                    posa = va + plsc.cumsum(il) - 1
                    posb = vb + plsc.cumsum(mh.astype(jnp.int32)) - 1
                    plsc.store_scatter(keys_v, [posa], iv, mask=ml)
                    plsc.store_scatter(keys_v, [posb], iv, mask=mh)
                    return (
                        accv + il,
                        va + plsc.all_reduce_population_count(ml),
                        vb + plsc.all_reduce_population_count(mh),
                    )

                acc = plsc.parallel_loop(
                    0, nv, unroll=2,
                    carry=(zero16,
                           jnp.full((L,), ab, jnp.int32),
                           jnp.full((L,), bb, jnp.int32)),
                )(level_pass)
                c = jnp.sum(acc[0])
                low = k < c
                p2 = lax.select(low, p, cand)
                k2 = lax.select(low, k, k - c)
                n2 = lax.select(low, c, n - c)
                sb2 = lax.select(low, ab, bb)
                bb2 = lax.select(low, bb, ab)
                return p2, k2, n2, sb2, sb, bb2, b - 1

            p, k, n, sb, _, _, b = lax.while_loop(
                level_cond, level, (p, k, n, sb0, ab0, bb0, b0)
            )

            def tail():
                iv = keys_v[pl.ds(sb, L)]
                ks = iv ^ ((iv >> 31) & jnp.int32(0x7FFFFFFF))
                ks = jnp.where(lane < n, ks, jnp.int32(0x7FFFFFFF))
                srt = lax.sort(ks)
                kv = jnp.take_along_axis(
                    srt, jnp.full((L,), k, jnp.int32), axis=0,
                    mode="promise_in_bounds",
                )
                return plsc.bitcast(
                    kv ^ ((kv >> 31) & jnp.int32(0x7FFFFFFF)), jnp.float32
                )

            def from_prefix():
                pos = p < jnp.int32(0)
                fbits = lax.select(pos, p ^ int_min, ~p)
                return jnp.full(
                    (L,), lax.bitcast_convert_type(fbits, jnp.float32)
                )

            val = lax.cond(n <= L, tail, from_prefix)

            plsc.store_scatter(
                res_v,
                [jnp.full((L,), r, jnp.int32)],
                val,
                mask=lane == jnp.int32(0),
            )
            return carry

        lax.fori_loop(0, RPW, row_body, jnp.int32(0))
        pltpu.sync_copy(res_v, out_hbm.at[pl.ds(wid * RPW, RPW)])

    return sc_kernel


_sc_kernel = _make_sc_kernel()


def kernel(x):
    B0, B1, n = x.shape
    flat = _sc_kernel(x.reshape(B0 * B1, n))
    return flat.reshape(B0, B1)


# speedup vs baseline: 7.3111x; 1.0797x over previous
import functools

import jax
import jax.numpy as jnp
from jax import lax
from jax.experimental import pallas as pl
from jax.experimental.pallas import tpu as pltpu
from jax.experimental.pallas import tpu_sc as plsc

R = 2048
N = 8192
K = N - 1 - N // 2
NW = 32
RPW = R // NW
L = 16
NV = N // L
U = 4
US = 2
NB = 256
SH = 24
HIST_W = US * L * NB

_INT_MIN_PY = -2147483648


def _make_sc_kernel():
    mesh = plsc.VectorSubcoreMesh(core_axis_name="c", subcore_axis_name="s")

    @functools.partial(
        pl.kernel,
        mesh=mesh,
        compiler_params=pltpu.CompilerParams(needs_layout_passes=False),
        out_type=jax.ShapeDtypeStruct((R,), jnp.float32),
        scratch_types=[
            pltpu.VMEM((2 * N,), jnp.float32),
            pltpu.VMEM((3 * N,), jnp.int32),
            pltpu.VMEM((HIST_W,), jnp.int32),
            pltpu.VMEM((RPW,), jnp.float32),
            pltpu.SemaphoreType.DMA,
        ],
    )
    def sc_kernel(x_hbm, out_hbm, rows_v, keys_v, hist_v, res_v, sem):
        wid = lax.axis_index("c") * 16 + lax.axis_index("s")
        base0 = wid * RPW
        lane = lax.iota(jnp.int32, L)
        zero16 = jnp.zeros((L,), jnp.int32)
        ones16 = jnp.ones((L,), jnp.int32)
        int_min = jnp.int32(_INT_MIN_PY)
        lane_nb = lane * NB

        @plsc.parallel_loop(0, HIST_W // L, unroll=4)
        def _(i):
            hist_v[pl.ds(i * L, L)] = zero16

        pltpu.async_copy(x_hbm.at[base0], rows_v.at[pl.ds(0, N)], sem)

        def load_key(off):
            xv = rows_v[pl.ds(off, L)]
            iv = plsc.bitcast(xv, jnp.int32)
            return iv ^ ((iv >> 31) & jnp.int32(0x7FFFFFFF))

        def row_body(r, carry):
            nxt = r + 1

            @pl.when(nxt < RPW)
            def _():
                pltpu.async_copy(
                    x_hbm.at[base0 + nxt],
                    rows_v.at[pl.ds((nxt % 2) * N, N)],
                    sem,
                )

            pltpu.make_async_copy(
                x_hbm.at[base0], rows_v.at[pl.ds(0, N)], sem
            ).wait()
            rb = (r % 2) * N

            @plsc.parallel_loop(0, NV, unroll=U)
            def _(i):
                ks = load_key(rb + i * L)
                du = ((ks ^ int_min) >> SH) & jnp.int32(NB - 1)
                idx = lane_nb + (i % US) * (L * NB) + du
                plsc.addupdate_scatter(hist_v, [idx], ones16)

            NG = NB // L

            def tot_body(s_, tc):
                outs = []
                for i in range(NG):
                    sl = pl.ds(s_ * NB + i * L, L)
                    outs.append(tc[i] + hist_v[sl])
                    hist_v[sl] = zero16
                return tuple(outs)

            tot = plsc.parallel_loop(
                0, US * L, unroll=1, carry=(zero16,) * NG
            )(tot_body)

            kk = jnp.int32(K)
            t = [jnp.sum(tot[i]) for i in range(NG)]
            cums = []
            run = t[0]
            for i in range(1, NG):
                cums.append(run)
                run = run + t[i]
            i_star = jnp.int32(0)
            for c in cums:
                i_star = i_star + (kk >= c).astype(jnp.int32)
            tb = jnp.int32(0)
            for c in cums:
                tb = jnp.where(kk >= c, c, tb)
            pv = jnp.full((L,), i_star, jnp.int32)
            tot_sel = tot[NG - 1]
            for i in range(NG - 2, -1, -1):
                tot_sel = jnp.where(pv == i, tot[i], tot_sel)
            cum = plsc.cumsum(tot_sel) + tb
            mle = cum <= kk
            d_vec = plsc.all_reduce_population_count(mle) + i_star * L
            cum_before = jnp.max(jnp.where(mle, cum, tb))
            cum_d = jnp.min(jnp.where(mle, jnp.int32(1 << 30), cum))
            n = cum_d - cum_before
            k = kk - cum_before
            d_scalar = jnp.max(d_vec)
            p = lax.shift_left(d_scalar, jnp.int32(SH))

            dge = d_scalar >= jnp.int32(NB // 2)
            a_s = jnp.where(
                dge,
                lax.shift_left(d_scalar - jnp.int32(NB // 2), jnp.int32(SH)),
                -lax.shift_left(d_scalar + jnp.int32(1), jnp.int32(SH)),
            )
            b_s = jnp.where(
                dge,
                lax.shift_left(d_scalar - jnp.int32(NB // 2 - 1), jnp.int32(SH)),
                -lax.shift_left(d_scalar, jnp.int32(SH)),
            )
            a_v = jnp.full((L,), a_s, jnp.int32)
            b_v = jnp.full((L,), b_s, jnp.int32)

            def compact_body(i, vw):
                xv = rows_v[pl.ds(rb + i * L, L)]
                iv = plsc.bitcast(xv, jnp.int32)
                m = (iv >= a_v) & (iv < b_v)
                pos = vw + plsc.cumsum(m.astype(jnp.int32)) - 1
                plsc.store_scatter(keys_v, [pos], iv, mask=m)
                return vw + plsc.all_reduce_population_count(m)

            plsc.parallel_loop(0, NV, unroll=U, carry=zero16)(compact_body)

            sb0 = jnp.int32(0)
            ab0 = jnp.int32(N)
            bb0 = jnp.int32(2 * N)
            b0 = jnp.int32(SH - 1)

            def level_cond(bc):
                _, _, n, _, _, _, b = bc
                return (n > L) & (b >= 0)

            def level(bc):
                p, k, n, sb, ab, bb, b = bc
                cand = p | lax.shift_left(jnp.int32(1), b)
                ccmp = cand ^ int_min
                nv = (n + L - 1) // L

                def level_pass(j, vc):
                    accv, va, vb = vc
                    iv = keys_v[pl.ds(sb + j * L, L)]
                    ks = iv ^ ((iv >> 31) & jnp.int32(0x7FFFFFFF))
                    valid = (j * L + lane) < n
                    ml = (ks < ccmp) & valid
                    mh = valid & ~ml
                    il = ml.astype(jnp.int32)
                    posa = va + plsc.cumsum(il) - 1
                    posb = vb + plsc.cumsum(mh.astype(jnp.int32)) - 1
                    plsc.store_scatter(keys_v, [posa], iv, mask=ml)
                    plsc.store_scatter(keys_v, [posb], iv, mask=mh)
                    return (
                        accv + il,
                        va + plsc.all_reduce_population_count(ml),
                        vb + plsc.all_reduce_population_count(mh),
                    )

                acc = plsc.parallel_loop(
                    0, nv, unroll=2,
                    carry=(zero16,
                           jnp.full((L,), ab, jnp.int32),
                           jnp.full((L,), bb, jnp.int32)),
                )(level_pass)
                c = jnp.sum(acc[0])
                low = k < c
                p2 = lax.select(low, p, cand)
                k2 = lax.select(low, k, k - c)
                n2 = lax.select(low, c, n - c)
                sb2 = lax.select(low, ab, bb)
                bb2 = lax.select(low, bb, ab)
                return p2, k2, n2, sb2, sb, bb2, b - 1

            p, k, n, sb, _, _, b = lax.while_loop(
                level_cond, level, (p, k, n, sb0, ab0, bb0, b0)
            )

            def tail():
                iv = keys_v[pl.ds(sb, L)]
                ks = iv ^ ((iv >> 31) & jnp.int32(0x7FFFFFFF))
                ks = jnp.where(lane < n, ks, jnp.int32(0x7FFFFFFF))
                srt = lax.sort(ks)
                kv = jnp.take_along_axis(
                    srt, jnp.full((L,), k, jnp.int32), axis=0,
                    mode="promise_in_bounds",
                )
                return plsc.bitcast(
                    kv ^ ((kv >> 31) & jnp.int32(0x7FFFFFFF)), jnp.float32
                )

            def from_prefix():
                pos = p < jnp.int32(0)
                fbits = lax.select(pos, p ^ int_min, ~p)
                return jnp.full(
                    (L,), lax.bitcast_convert_type(fbits, jnp.float32)
                )

            val = lax.cond(n <= L, tail, from_prefix)

            plsc.store_scatter(
                res_v,
                [jnp.full((L,), r, jnp.int32)],
                val,
                mask=lane == jnp.int32(0),
            )
            return carry

        lax.fori_loop(0, RPW, row_body, jnp.int32(0))
        pltpu.sync_copy(res_v, out_hbm.at[pl.ds(wid * RPW, RPW)])

    return sc_kernel


_sc_kernel = _make_sc_kernel()


def kernel(x):
    B0, B1, n = x.shape
    flat = _sc_kernel(x.reshape(B0 * B1, n))
    return flat.reshape(B0, B1)


# speedup vs baseline: 7.6169x; 1.0418x over previous
import functools

import jax
import jax.numpy as jnp
from jax import lax
from jax.experimental import pallas as pl
from jax.experimental.pallas import tpu as pltpu
from jax.experimental.pallas import tpu_sc as plsc

R = 2048
N = 8192
K = N - 1 - N // 2
NW = 32
RPW = R // NW
L = 16
NV = N // L
U = 4
US = 1
NB = 256
SH = 24
HIST_W = US * L * NB

_INT_MIN_PY = -2147483648


def _make_sc_kernel():
    mesh = plsc.VectorSubcoreMesh(core_axis_name="c", subcore_axis_name="s")

    @functools.partial(
        pl.kernel,
        mesh=mesh,
        compiler_params=pltpu.CompilerParams(needs_layout_passes=False),
        out_type=jax.ShapeDtypeStruct((R,), jnp.float32),
        scratch_types=[
            pltpu.VMEM((2 * N,), jnp.float32),
            pltpu.VMEM((3 * N,), jnp.int32),
            pltpu.VMEM((HIST_W,), jnp.int32),
            pltpu.VMEM((RPW,), jnp.float32),
            pltpu.SemaphoreType.DMA,
        ],
    )
    def sc_kernel(x_hbm, out_hbm, rows_v, keys_v, hist_v, res_v, sem):
        wid = lax.axis_index("c") * 16 + lax.axis_index("s")
        base0 = wid * RPW
        lane = lax.iota(jnp.int32, L)
        zero16 = jnp.zeros((L,), jnp.int32)
        ones16 = jnp.ones((L,), jnp.int32)
        int_min = jnp.int32(_INT_MIN_PY)
        lane_nb = lane * NB

        @plsc.parallel_loop(0, HIST_W // L, unroll=4)
        def _(i):
            hist_v[pl.ds(i * L, L)] = zero16

        pltpu.async_copy(x_hbm.at[base0], rows_v.at[pl.ds(0, N)], sem)

        def load_key(off):
            xv = rows_v[pl.ds(off, L)]
            iv = plsc.bitcast(xv, jnp.int32)
            return iv ^ ((iv >> 31) & jnp.int32(0x7FFFFFFF))

        def row_body(r, carry):
            nxt = r + 1

            @pl.when(nxt < RPW)
            def _():
                pltpu.async_copy(
                    x_hbm.at[base0 + nxt],
                    rows_v.at[pl.ds((nxt % 2) * N, N)],
                    sem,
                )

            pltpu.make_async_copy(
                x_hbm.at[base0], rows_v.at[pl.ds(0, N)], sem
            ).wait()
            rb = (r % 2) * N

            @plsc.parallel_loop(0, NV, unroll=U)
            def _(i):
                ks = load_key(rb + i * L)
                du = ((ks ^ int_min) >> SH) & jnp.int32(NB - 1)
                idx = lane_nb + du
                plsc.addupdate_scatter(hist_v, [idx], ones16)

            NG = NB // L

            def tot_body(s_, tc):
                outs = []
                for i in range(NG):
                    sl = pl.ds(s_ * NB + i * L, L)
                    outs.append(tc[i] + hist_v[sl])
                    hist_v[sl] = zero16
                return tuple(outs)

            tot = plsc.parallel_loop(
                0, US * L, unroll=1, carry=(zero16,) * NG
            )(tot_body)

            kk = jnp.int32(K)
            t = [jnp.sum(tot[i]) for i in range(NG)]
            cums = []
            run = t[0]
            for i in range(1, NG):
                cums.append(run)
                run = run + t[i]
            i_star = jnp.int32(0)
            for c in cums:
                i_star = i_star + (kk >= c).astype(jnp.int32)
            tb = jnp.int32(0)
            for c in cums:
                tb = jnp.where(kk >= c, c, tb)
            pv = jnp.full((L,), i_star, jnp.int32)
            tot_sel = tot[NG - 1]
            for i in range(NG - 2, -1, -1):
                tot_sel = jnp.where(pv == i, tot[i], tot_sel)
            cum = plsc.cumsum(tot_sel) + tb
            mle = cum <= kk
            d_vec = plsc.all_reduce_population_count(mle) + i_star * L
            cum_before = jnp.max(jnp.where(mle, cum, tb))
            cum_d = jnp.min(jnp.where(mle, jnp.int32(1 << 30), cum))
            n = cum_d - cum_before
            k = kk - cum_before
            d_scalar = jnp.max(d_vec)
            p = lax.shift_left(d_scalar, jnp.int32(SH))

            dge = d_scalar >= jnp.int32(NB // 2)
            a_s = jnp.where(
                dge,
                lax.shift_left(d_scalar - jnp.int32(NB // 2), jnp.int32(SH)),
                -lax.shift_left(d_scalar + jnp.int32(1), jnp.int32(SH)),
            )
            b_s = jnp.where(
                dge,
                lax.shift_left(d_scalar - jnp.int32(NB // 2 - 1), jnp.int32(SH)),
                -lax.shift_left(d_scalar, jnp.int32(SH)),
            )
            a_v = jnp.full((L,), a_s, jnp.int32)
            b_v = jnp.full((L,), b_s, jnp.int32)

            def compact_body(i, vw):
                xv = rows_v[pl.ds(rb + i * L, L)]
                iv = plsc.bitcast(xv, jnp.int32)
                m = (iv >= a_v) & (iv < b_v)
                pos = vw + plsc.cumsum(m.astype(jnp.int32)) - 1
                plsc.store_scatter(keys_v, [pos], iv, mask=m)
                return vw + plsc.all_reduce_population_count(m)

            plsc.parallel_loop(0, NV, unroll=U, carry=zero16)(compact_body)

            sb0 = jnp.int32(0)
            ab0 = jnp.int32(N)
            bb0 = jnp.int32(2 * N)
            b0 = jnp.int32(SH - 1)

            def level_cond(bc):
                _, _, n, _, _, _, b = bc
                return (n > L) & (b >= 0)

            def level(bc):
                p, k, n, sb, ab, bb, b = bc
                cand = p | lax.shift_left(jnp.int32(1), b)
                ccmp = cand ^ int_min
                nv = (n + L - 1) // L

                def level_pass(j, vc):
                    accv, va, vb = vc
                    iv = keys_v[pl.ds(sb + j * L, L)]
                    ks = iv ^ ((iv >> 31) & jnp.int32(0x7FFFFFFF))
                    valid = (j * L + lane) < n
                    ml = (ks < ccmp) & valid
                    mh = valid & ~ml
                    il = ml.astype(jnp.int32)
                    posa = va + plsc.cumsum(il) - 1
                    posb = vb + plsc.cumsum(mh.astype(jnp.int32)) - 1
                    plsc.store_scatter(keys_v, [posa], iv, mask=ml)
                    plsc.store_scatter(keys_v, [posb], iv, mask=mh)
                    return (
                        accv + il,
                        va + plsc.all_reduce_population_count(ml),
                        vb + plsc.all_reduce_population_count(mh),
                    )

                acc = plsc.parallel_loop(
                    0, nv, unroll=2,
                    carry=(zero16,
                           jnp.full((L,), ab, jnp.int32),
                           jnp.full((L,), bb, jnp.int32)),
                )(level_pass)
                c = jnp.sum(acc[0])
                low = k < c
                p2 = lax.select(low, p, cand)
                k2 = lax.select(low, k, k - c)
                n2 = lax.select(low, c, n - c)
                sb2 = lax.select(low, ab, bb)
                bb2 = lax.select(low, bb, ab)
                return p2, k2, n2, sb2, sb, bb2, b - 1

            p, k, n, sb, _, _, b = lax.while_loop(
                level_cond, level, (p, k, n, sb0, ab0, bb0, b0)
            )

            def tail():
                iv = keys_v[pl.ds(sb, L)]
                ks = iv ^ ((iv >> 31) & jnp.int32(0x7FFFFFFF))
                ks = jnp.where(lane < n, ks, jnp.int32(0x7FFFFFFF))
                srt = lax.sort(ks)
                kv = jnp.take_along_axis(
                    srt, jnp.full((L,), k, jnp.int32), axis=0,
                    mode="promise_in_bounds",
                )
                return plsc.bitcast(
                    kv ^ ((kv >> 31) & jnp.int32(0x7FFFFFFF)), jnp.float32
                )

            def from_prefix():
                pos = p < jnp.int32(0)
                fbits = lax.select(pos, p ^ int_min, ~p)
                return jnp.full(
                    (L,), lax.bitcast_convert_type(fbits, jnp.float32)
                )

            val = lax.cond(n <= L, tail, from_prefix)

            plsc.store_scatter(
                res_v,
                [jnp.full((L,), r, jnp.int32)],
                val,
                mask=lane == jnp.int32(0),
            )
            return carry

        lax.fori_loop(0, RPW, row_body, jnp.int32(0))
        pltpu.sync_copy(res_v, out_hbm.at[pl.ds(wid * RPW, RPW)])

    return sc_kernel


_sc_kernel = _make_sc_kernel()


def kernel(x):
    B0, B1, n = x.shape
    flat = _sc_kernel(x.reshape(B0 * B1, n))
    return flat.reshape(B0, B1)
